# baseline (speedup 1.0000x reference)
import jax
import jax.numpy as jnp
from jax import lax
from jax.experimental import pallas as pl
from jax.experimental.pallas import tpu as pltpu
from jax.experimental.pallas import tpu_sc as plsc

BATCH = 4
CTX = 8192
EMB = 1024
BLK = 512

NC, NS = 2, 16
NW = NC * NS
SC_BATCH = 3
RPW = CTX // NW
CHUNK = 16
CELTS = CHUNK * EMB
NCHUNK = RPW // CHUNK


def _sc_body(x_hbm, pos_hbm, o_hbm,
             xv0, xv1, pv0, pv1,
             sx0, sx1, sp0, sp1, so0, so1):
    wid = lax.axis_index("s") * NC + lax.axis_index("c")
    obase = wid * (RPW * EMB)
    base = SC_BATCH * (CTX * EMB) + obase
    pbase = obase

    xv = (xv0, xv1)
    pv = (pv0, pv1)
    sx = (sx0, sx1)
    sp = (sp0, sp1)
    so = (so0, so1)

    def start_in(k, b):
        pltpu.async_copy(x_hbm.at[pl.ds(base + k * CELTS, CELTS)], xv[b], sx[b])
        pltpu.async_copy(pos_hbm.at[pl.ds(pbase + k * CELTS, CELTS)], pv[b], sp[b])

    start_in(0, 0)

    @pl.loop(0, NCHUNK, step=2)
    def _(k):
        for b in range(2):
            kk = k + b
            pltpu.make_async_copy(x_hbm.at[pl.ds(0, CELTS)], xv[b], sx[b]).wait()
            pltpu.make_async_copy(pos_hbm.at[pl.ds(0, CELTS)], pv[b], sp[b]).wait()

            @pl.when(kk >= 1)
            def _():
                pltpu.make_async_copy(
                    xv[1 - b], o_hbm.at[pl.ds(0, CELTS)], so[1 - b]).wait()

            @pl.when(kk + 1 < NCHUNK)
            def _():
                start_in(kk + 1, 1 - b)

            @pl.loop(0, CELTS // 16, unroll=8)
            def _(j):
                sl = pl.ds(j * 16, 16)
                xv[b][sl] = xv[b][sl] + pv[b][sl]

            pltpu.async_copy(xv[b], o_hbm.at[pl.ds(obase + kk * CELTS, CELTS)],
                             so[b])

    pltpu.make_async_copy(
        xv[(NCHUNK - 1) % 2], o_hbm.at[pl.ds(0, CELTS)],
        so[(NCHUNK - 1) % 2]).wait()


def _sc_add(x_flat, pos_flat):
    mesh = plsc.VectorSubcoreMesh(core_axis_name="c", subcore_axis_name="s")
    return pl.kernel(
        _sc_body,
        out_type=jax.ShapeDtypeStruct((CTX * EMB,), jnp.float32),
        mesh=mesh,
        scratch_types=[
            pltpu.VMEM((CELTS,), jnp.float32),
            pltpu.VMEM((CELTS,), jnp.float32),
            pltpu.VMEM((CELTS,), jnp.float32),
            pltpu.VMEM((CELTS,), jnp.float32),
            pltpu.SemaphoreType.DMA,
            pltpu.SemaphoreType.DMA,
            pltpu.SemaphoreType.DMA,
            pltpu.SemaphoreType.DMA,
            pltpu.SemaphoreType.DMA,
            pltpu.SemaphoreType.DMA,
        ],
    )(x_flat, pos_flat)


def _tc_kernel(x_ref, pos_ref, o_ref):
    o_ref[...] = x_ref[...] + pos_ref[...][None, :, :]


def _tc_add(x, pos_table):
    return pl.pallas_call(
        _tc_kernel,
        grid=(CTX // BLK, BATCH - 1),
        in_specs=[
            pl.BlockSpec((1, BLK, EMB), lambda i, b: (b, i, 0)),
            pl.BlockSpec((BLK, EMB), lambda i, b: (i, 0)),
        ],
        out_specs=pl.BlockSpec((1, BLK, EMB), lambda i, b: (b, i, 0)),
        out_shape=jax.ShapeDtypeStruct((BATCH - 1, CTX, EMB), x.dtype),
    )(x, pos_table)


@jax.jit
def _hybrid(x, pos_table):
    sc_out = _sc_add(x.reshape(-1), pos_table.reshape(-1))
    tc_out = _tc_add(x, pos_table)
    return jnp.concatenate([tc_out, sc_out.reshape(1, CTX, EMB)], axis=0)


def kernel(x, pos_table):
    return _hybrid(x, pos_table)


# speedup vs baseline: 3.9417x; 3.9417x over previous
import jax
import jax.numpy as jnp
from jax.experimental import pallas as pl

BLK = 512


def _add_kernel(x_ref, pos_ref, o_ref):
    o_ref[...] = x_ref[...] + pos_ref[...][None, :, :]


def kernel(x, pos_table):
    batch, ctx, emb = x.shape
    grid = (ctx // BLK,)
    return pl.pallas_call(
        _add_kernel,
        grid=grid,
        in_specs=[
            pl.BlockSpec((batch, BLK, emb), lambda i: (0, i, 0)),
            pl.BlockSpec((BLK, emb), lambda i: (i, 0)),
        ],
        out_specs=pl.BlockSpec((batch, BLK, emb), lambda i: (0, i, 0)),
        out_shape=jax.ShapeDtypeStruct(x.shape, x.dtype),
    )(x, pos_table)
